# baseline (speedup 1.0000x reference)
import functools

import jax
import jax.numpy as jnp
from jax import lax
from jax.experimental import pallas as pl
from jax.experimental.pallas import tpu as pltpu
from jax.experimental.pallas import tpu_sc as plsc

N = 10000
F = 128
HOPS = 3
NC, NS = 2, 16
NT = NC * NS
NPAD = 10240
ROWS_PT = NPAD // NS
C = 128
E_REAL = 320000
E_LOOP = E_REAL + N
NBUF = 2
G = 8
NGRP = 12
NCH = G * NGRP
EPAD = NT * NCH * C
BM = 1024

_mesh = plsc.VectorSubcoreMesh(
    core_axis_name="c", subcore_axis_name="s", num_cores=NC, num_subcores=NS)


@functools.partial(
    pl.kernel,
    out_type=jax.ShapeDtypeStruct((NC, NPAD), jnp.float32),
    mesh=_mesh,
    scratch_types=[
        pltpu.VMEM((NCH, C), jnp.int32),
        pltpu.VMEM((C,), jnp.float32),
        pltpu.VMEM_SHARED((NPAD,), jnp.float32),
    ],
)
def _deg_kernel(col_hbm, ones_hbm, zero_hbm, out_hbm, idxc_v, ones_v, acc_s):
    cc = lax.axis_index("c")
    ss = lax.axis_index("s")
    t = ss * NC + cc
    pltpu.sync_copy(col_hbm.at[t], idxc_v)
    pltpu.sync_copy(ones_hbm, ones_v)
    base = ss * ROWS_PT
    pltpu.sync_copy(zero_hbm.at[pl.ds(base, ROWS_PT)], acc_s.at[pl.ds(base, ROWS_PT)])
    plsc.subcore_barrier()

    def body(j, carry):
        pltpu.sync_copy(ones_v, acc_s.at[idxc_v.at[j]], add=True)
        return carry

    lax.fori_loop(0, NCH, body, 0)
    plsc.subcore_barrier()
    pltpu.sync_copy(acc_s.at[pl.ds(base, ROWS_PT)],
                    out_hbm.at[cc, pl.ds(base, ROWS_PT)])


@functools.partial(
    pl.kernel,
    out_type=jax.ShapeDtypeStruct((NC, NPAD, F), jnp.float32),
    mesh=_mesh,
    scratch_types=[
        pltpu.VMEM((2, G, C), jnp.int32),
        pltpu.VMEM((2, G, C), jnp.int32),
        pltpu.VMEM((NBUF, C, F), jnp.float32),
        pltpu.VMEM_SHARED((NPAD, F), jnp.float32),
        pltpu.SemaphoreType.DMA((NBUF,)),
        pltpu.SemaphoreType.DMA((2,)),
    ],
)
def _hop_kernel(g_hbm, row_hbm, col_hbm, zrow_hbm, out_hbm,
                idxr_v, idxc_v, rows_v, acc_s, dsems, isems):
    cc = lax.axis_index("c")
    ss = lax.axis_index("s")
    t = ss * NC + cc

    def stage_group(g, slot):
        pltpu.async_copy(row_hbm.at[t, pl.ds(g * G, G)], idxr_v.at[slot],
                         isems.at[slot])
        pltpu.async_copy(col_hbm.at[t, pl.ds(g * G, G)], idxc_v.at[slot],
                         isems.at[slot])

    def wait_group(slot):
        pltpu.make_async_copy(row_hbm.at[t, pl.ds(0, G)], idxr_v.at[slot],
                              isems.at[slot]).wait()
        pltpu.make_async_copy(col_hbm.at[t, pl.ds(0, G)], idxc_v.at[slot],
                              isems.at[slot]).wait()

    stage_group(0, 0)
    stage_group(1, 1)
    base = ss * ROWS_PT
    pltpu.sync_copy(zrow_hbm.at[pl.ds(base, ROWS_PT)], acc_s.at[pl.ds(base, ROWS_PT)])
    wait_group(0)
    plsc.subcore_barrier()

    def pair_body(gi2, carry):
        for half in range(2):
            g = gi2 * 2 + half
            nslot = 1 - half

            @pl.when(g + 1 < NGRP)
            def _stage_next():
                stage_group(g + 1, nslot)

            for k in range(G):
                pltpu.async_copy(g_hbm.at[idxr_v.at[half, k]], rows_v.at[0],
                                 dsems.at[0]).wait()
                pltpu.sync_copy(rows_v.at[0], acc_s.at[idxc_v.at[half, k]],
                                add=True)

            @pl.when(g + 1 < NGRP)
            def _wait_next():
                wait_group(nslot)
        return carry

    lax.fori_loop(0, NGRP // 2, pair_body, 0)
    plsc.subcore_barrier()
    pltpu.sync_copy(acc_s.at[pl.ds(base, ROWS_PT)],
                    out_hbm.at[cc, pl.ds(base, ROWS_PT)])


def _dinv_body(d_ref, o1_ref, o2_ref):
    deg = d_ref[:, 0:1] + d_ref[:, 1:2]
    dinv = jnp.where(deg > 0.0, lax.rsqrt(deg), 0.0)
    o1_ref[...] = dinv
    o2_ref[...] = dinv * dinv


def _mm_body(x_ref, wt_ref, b_ref, s_ref, o_ref):
    h = jnp.dot(x_ref[...], wt_ref[...], preferred_element_type=jnp.float32)
    o_ref[...] = s_ref[...] * (h + b_ref[...])


def _scale_body(a_ref, s_ref, o_ref):
    o_ref[...] = s_ref[...] * (a_ref[0] + a_ref[1])


def _dinv_call(deg_t):
    return pl.pallas_call(
        _dinv_body,
        grid=(NPAD // BM,),
        in_specs=[pl.BlockSpec((BM, NC), lambda i: (i, 0))],
        out_specs=[pl.BlockSpec((BM, 1), lambda i: (i, 0)),
                   pl.BlockSpec((BM, 1), lambda i: (i, 0))],
        out_shape=[jax.ShapeDtypeStruct((NPAD, 1), jnp.float32),
                   jax.ShapeDtypeStruct((NPAD, 1), jnp.float32)],
    )(deg_t)


def _mm_call(xpad, wt, b2, sv):
    return pl.pallas_call(
        _mm_body,
        grid=(NPAD // BM,),
        in_specs=[pl.BlockSpec((BM, F), lambda i: (i, 0)),
                  pl.BlockSpec((F, F), lambda i: (0, 0)),
                  pl.BlockSpec((1, F), lambda i: (0, 0)),
                  pl.BlockSpec((BM, 1), lambda i: (i, 0))],
        out_specs=pl.BlockSpec((BM, F), lambda i: (i, 0)),
        out_shape=jax.ShapeDtypeStruct((NPAD, F), jnp.float32),
    )(xpad, wt, b2, sv)


def _scale_call(acc_pair, sv):
    return pl.pallas_call(
        _scale_body,
        grid=(NPAD // BM,),
        in_specs=[pl.BlockSpec((NC, BM, F), lambda i: (0, i, 0)),
                  pl.BlockSpec((BM, 1), lambda i: (i, 0))],
        out_specs=pl.BlockSpec((BM, F), lambda i: (i, 0)),
        out_shape=jax.ShapeDtypeStruct((NPAD, F), jnp.float32),
    )(acc_pair, sv)


def kernel(x, edge_index, W, b):
    ei = edge_index.astype(jnp.int32)
    loop = jnp.arange(N, dtype=jnp.int32)
    pad_e = EPAD - E_LOOP
    row = jnp.concatenate([ei[0], loop, jnp.zeros((pad_e,), jnp.int32)])
    col = jnp.concatenate([ei[1], loop, jnp.full((pad_e,), N, jnp.int32)])
    row3 = row.reshape(NT, NCH, C)
    col3 = col.reshape(NT, NCH, C)

    xpad = jnp.zeros((NPAD, F), jnp.float32).at[:N].set(x)
    wt = W.T
    b2 = b.reshape(1, F)
    ones_c = jnp.ones((C,), jnp.float32)
    zero_n = jnp.zeros((NPAD,), jnp.float32)
    zero_rows = jnp.zeros((NPAD, F), jnp.float32)

    deg_pair = _deg_kernel(col3, ones_c, zero_n)
    dinv, dinv2 = _dinv_call(deg_pair.T)

    g = _mm_call(xpad, wt, b2, dinv)
    for hop in range(HOPS):
        acc_pair = _hop_kernel(g, row3, col3, zero_rows)
        sv = dinv if hop == HOPS - 1 else dinv2
        g = _scale_call(acc_pair, sv)
    return g[:N]


# speedup vs baseline: 3.6615x; 3.6615x over previous
import functools

import jax
import jax.numpy as jnp
from jax import lax
from jax.experimental import pallas as pl
from jax.experimental.pallas import tpu as pltpu
from jax.experimental.pallas import tpu_sc as plsc

N = 10000
F = 128
HOPS = 3
NC, NS = 2, 16
NT = NC * NS
NPAD = 10240
ROWS_PT = NPAD // NS
C = 128
E_REAL = 320000
E_LOOP = E_REAL + N
NBUF = 2
NCH = 82
NCHJ = NCH + NBUF
EPAD = NT * NCH * C
JCOL = N + 16
BM = 1024

_mesh = plsc.VectorSubcoreMesh(
    core_axis_name="c", subcore_axis_name="s", num_cores=NC, num_subcores=NS)


def _unpack_col(pk_v, j, colb_v):
    for i in range(C // 16):
        v = pk_v[j, pl.ds(i * 16, 16)]
        colb_v[pl.ds(i * 16, 16)] = lax.bitwise_and(v, 0xFFFF)


def _unpack_row(pk_v, j, rowb_v, bb):
    for i in range(C // 16):
        v = pk_v[j, pl.ds(i * 16, 16)]
        rowb_v[bb, pl.ds(i * 16, 16)] = lax.shift_right_logical(v, 16)


@functools.partial(
    pl.kernel,
    out_type=jax.ShapeDtypeStruct((NC, NPAD), jnp.float32),
    mesh=_mesh,
    scratch_types=[
        pltpu.VMEM((NCHJ, C), jnp.int32),
        pltpu.VMEM((C,), jnp.int32),
        pltpu.VMEM((C,), jnp.float32),
        pltpu.VMEM_SHARED((NPAD,), jnp.float32),
    ],
)
def _deg_kernel(pk_hbm, ones_hbm, zero_hbm, out_hbm, pk_v, colb_v, ones_v, acc_s):
    cc = lax.axis_index("c")
    ss = lax.axis_index("s")
    t = ss * NC + cc
    pltpu.sync_copy(pk_hbm.at[t], pk_v)
    pltpu.sync_copy(ones_hbm, ones_v)
    base = ss * ROWS_PT
    pltpu.sync_copy(zero_hbm.at[pl.ds(base, ROWS_PT)], acc_s.at[pl.ds(base, ROWS_PT)])
    plsc.subcore_barrier()

    def body(j, carry):
        _unpack_col(pk_v, j, colb_v)
        pltpu.sync_copy(ones_v, acc_s.at[colb_v], add=True)
        return carry

    lax.fori_loop(0, NCH, body, 0)
    plsc.subcore_barrier()
    pltpu.sync_copy(acc_s.at[pl.ds(base, ROWS_PT)],
                    out_hbm.at[cc, pl.ds(base, ROWS_PT)])


@functools.partial(
    pl.kernel,
    out_type=jax.ShapeDtypeStruct((NC, NPAD, F), jnp.float32),
    mesh=_mesh,
    scratch_types=[
        pltpu.VMEM((NCHJ, C), jnp.int32),
        pltpu.VMEM((NBUF, C), jnp.int32),
        pltpu.VMEM((C,), jnp.int32),
        pltpu.VMEM((NBUF, C, F), jnp.float32),
        pltpu.VMEM_SHARED((NPAD, F), jnp.float32),
        pltpu.SemaphoreType.DMA((NBUF,)),
    ],
)
def _hop_kernel(g_hbm, pk_hbm, zrow_hbm, out_hbm,
                pk_v, rowb_v, colb_v, rows_v, acc_s, dsems):
    cc = lax.axis_index("c")
    ss = lax.axis_index("s")
    t = ss * NC + cc

    pltpu.sync_copy(pk_hbm.at[t], pk_v)
    base = ss * ROWS_PT
    pltpu.sync_copy(zrow_hbm.at[pl.ds(base, ROWS_PT)], acc_s.at[pl.ds(base, ROWS_PT)])
    for bb in range(NBUF):
        _unpack_row(pk_v, bb, rowb_v, bb)
        pltpu.async_copy(g_hbm.at[rowb_v.at[bb]], rows_v.at[bb], dsems.at[bb])
    plsc.subcore_barrier()

    def pair_body(j2, carry):
        for bb in range(NBUF):
            j = j2 * NBUF + bb
            pltpu.make_async_copy(
                g_hbm.at[rowb_v.at[bb]], rows_v.at[bb], dsems.at[bb]).wait()
            _unpack_col(pk_v, j, colb_v)
            pltpu.sync_copy(rows_v.at[bb], acc_s.at[colb_v], add=True)
            _unpack_row(pk_v, j + NBUF, rowb_v, bb)
            pltpu.async_copy(g_hbm.at[rowb_v.at[bb]], rows_v.at[bb], dsems.at[bb])
        return carry

    lax.fori_loop(0, NCH // NBUF, pair_body, 0)
    for bb in range(NBUF):
        pltpu.make_async_copy(
            g_hbm.at[rowb_v.at[bb]], rows_v.at[bb], dsems.at[bb]).wait()
    plsc.subcore_barrier()
    pltpu.sync_copy(acc_s.at[pl.ds(base, ROWS_PT)],
                    out_hbm.at[cc, pl.ds(base, ROWS_PT)])


def _dinv_body(d_ref, o1_ref, o2_ref):
    deg = d_ref[:, 0:1] + d_ref[:, 1:2]
    dinv = jnp.where(deg > 0.0, lax.rsqrt(deg), 0.0)
    o1_ref[...] = dinv
    o2_ref[...] = dinv * dinv


def _mm_body(x_ref, wt_ref, b_ref, s_ref, o_ref):
    h = jnp.dot(x_ref[...], wt_ref[...], preferred_element_type=jnp.float32)
    o_ref[...] = s_ref[...] * (h + b_ref[...])


def _scale_body(a_ref, s_ref, o_ref):
    o_ref[...] = s_ref[...] * (a_ref[0] + a_ref[1])


def _dinv_call(deg_t):
    return pl.pallas_call(
        _dinv_body,
        grid=(NPAD // BM,),
        in_specs=[pl.BlockSpec((BM, NC), lambda i: (i, 0))],
        out_specs=[pl.BlockSpec((BM, 1), lambda i: (i, 0)),
                   pl.BlockSpec((BM, 1), lambda i: (i, 0))],
        out_shape=[jax.ShapeDtypeStruct((NPAD, 1), jnp.float32),
                   jax.ShapeDtypeStruct((NPAD, 1), jnp.float32)],
    )(deg_t)


def _mm_call(xpad, wt, b2, sv):
    return pl.pallas_call(
        _mm_body,
        grid=(NPAD // BM,),
        in_specs=[pl.BlockSpec((BM, F), lambda i: (i, 0)),
                  pl.BlockSpec((F, F), lambda i: (0, 0)),
                  pl.BlockSpec((1, F), lambda i: (0, 0)),
                  pl.BlockSpec((BM, 1), lambda i: (i, 0))],
        out_specs=pl.BlockSpec((BM, F), lambda i: (i, 0)),
        out_shape=jax.ShapeDtypeStruct((NPAD, F), jnp.float32),
    )(xpad, wt, b2, sv)


def _scale_call(acc_pair, sv):
    return pl.pallas_call(
        _scale_body,
        grid=(NPAD // BM,),
        in_specs=[pl.BlockSpec((NC, BM, F), lambda i: (0, i, 0)),
                  pl.BlockSpec((BM, 1), lambda i: (i, 0))],
        out_specs=pl.BlockSpec((BM, F), lambda i: (i, 0)),
        out_shape=jax.ShapeDtypeStruct((NPAD, F), jnp.float32),
    )(acc_pair, sv)


def kernel(x, edge_index, W, b):
    ei = edge_index.astype(jnp.int32)
    loop = jnp.arange(N, dtype=jnp.int32)
    pad_e = EPAD - E_LOOP
    row = jnp.concatenate([ei[0], loop, jnp.zeros((pad_e,), jnp.int32)])
    col = jnp.concatenate([ei[1], loop, jnp.full((pad_e,), JCOL, jnp.int32)])
    packed = jnp.left_shift(row, 16) | col
    pk3 = packed.reshape(NT, NCH, C)
    junk = jnp.full((NT, NBUF, C), JCOL, jnp.int32)
    pk3 = jnp.concatenate([pk3, junk], axis=1)

    xpad = jnp.zeros((NPAD, F), jnp.float32).at[:N].set(x)
    wt = W.T
    b2 = b.reshape(1, F)
    ones_c = jnp.ones((C,), jnp.float32)
    zero_n = jnp.zeros((NPAD,), jnp.float32)
    zero_rows = jnp.zeros((NPAD, F), jnp.float32)

    deg_pair = _deg_kernel(pk3, ones_c, zero_n)
    dinv, dinv2 = _dinv_call(deg_pair.T)

    g = _mm_call(xpad, wt, b2, dinv)
    for hop in range(HOPS):
        acc_pair = _hop_kernel(g, pk3, zero_rows)
        sv = dinv if hop == HOPS - 1 else dinv2
        g = _scale_call(acc_pair, sv)
    return g[:N]


# speedup vs baseline: 8.8296x; 2.4114x over previous
import functools

import jax
import jax.numpy as jnp
from jax import lax
from jax.experimental import pallas as pl
from jax.experimental.pallas import tpu as pltpu
from jax.experimental.pallas import tpu_sc as plsc

N = 10000
F = 128
HOPS = 3
NC, NS = 2, 16
NT = NC * NS
NPAD = 10240
ROWS_PT = NPAD // NS
C = 128
E_REAL = 320000
E_LOOP = E_REAL + N
NCH = 81
EPAD = NT * NCH * C
JCOL = N + 16
BM = 1024

_mesh = plsc.VectorSubcoreMesh(
    core_axis_name="c", subcore_axis_name="s", num_cores=NC, num_subcores=NS)


@functools.partial(
    pl.kernel,
    out_type=jax.ShapeDtypeStruct((NC, NPAD), jnp.float32),
    mesh=_mesh,
    scratch_types=[
        pltpu.VMEM((NCH, C), jnp.int32),
        pltpu.VMEM((C,), jnp.float32),
        pltpu.VMEM_SHARED((NPAD,), jnp.float32),
    ],
)
def _deg_kernel(col_hbm, ones_hbm, zero_hbm, out_hbm, idxc_v, ones_v, acc_s):
    cc = lax.axis_index("c")
    ss = lax.axis_index("s")
    t = ss * NC + cc
    pltpu.sync_copy(col_hbm.at[t], idxc_v)
    pltpu.sync_copy(ones_hbm, ones_v)
    base = ss * ROWS_PT
    pltpu.sync_copy(zero_hbm.at[pl.ds(base, ROWS_PT)], acc_s.at[pl.ds(base, ROWS_PT)])
    plsc.subcore_barrier()

    def body(j, carry):
        pltpu.sync_copy(ones_v, acc_s.at[idxc_v.at[j]], add=True)
        return carry

    lax.fori_loop(0, NCH, body, 0)
    plsc.subcore_barrier()
    pltpu.sync_copy(acc_s.at[pl.ds(base, ROWS_PT)],
                    out_hbm.at[cc, pl.ds(base, ROWS_PT)])


@functools.partial(
    pl.kernel,
    out_type=jax.ShapeDtypeStruct((NC, NPAD, F), jnp.float32),
    mesh=_mesh,
    scratch_types=[
        pltpu.VMEM((NCH, C), jnp.int32),
        pltpu.VMEM((NCH, C), jnp.int32),
        pltpu.VMEM((C, F), jnp.float32),
        pltpu.VMEM_SHARED((NPAD, F), jnp.float32),
        pltpu.SemaphoreType.DMA,
    ],
)
def _hop_kernel(g_hbm, row_hbm, col_hbm, zrow_hbm, out_hbm,
                idxr_v, idxc_v, rows_v, acc_s, sem):
    cc = lax.axis_index("c")
    ss = lax.axis_index("s")
    t = ss * NC + cc
    pltpu.sync_copy(row_hbm.at[t], idxr_v)
    pltpu.sync_copy(col_hbm.at[t], idxc_v)
    base = ss * ROWS_PT
    pltpu.sync_copy(zrow_hbm.at[pl.ds(base, ROWS_PT)], acc_s.at[pl.ds(base, ROWS_PT)])
    plsc.subcore_barrier()

    def body(j, carry):
        pltpu.async_copy(g_hbm.at[idxr_v.at[j]], rows_v, sem).wait()
        pltpu.sync_copy(rows_v, acc_s.at[idxc_v.at[j]], add=True)
        return carry

    lax.fori_loop(0, NCH, body, 0)
    plsc.subcore_barrier()
    pltpu.sync_copy(acc_s.at[pl.ds(base, ROWS_PT)],
                    out_hbm.at[cc, pl.ds(base, ROWS_PT)])


def _dinv_body(d_ref, o1_ref, o2_ref):
    deg = d_ref[:, 0:1] + d_ref[:, 1:2]
    dinv = jnp.where(deg > 0.0, lax.rsqrt(deg), 0.0)
    o1_ref[...] = dinv
    o2_ref[...] = dinv * dinv


def _mm_body(x_ref, wt_ref, b_ref, s_ref, o_ref):
    h = jnp.dot(x_ref[...], wt_ref[...], preferred_element_type=jnp.float32)
    o_ref[...] = s_ref[...] * (h + b_ref[...])


def _scale_body(a_ref, s_ref, o_ref):
    o_ref[...] = s_ref[...] * (a_ref[0] + a_ref[1])


def _dinv_call(deg_t):
    return pl.pallas_call(
        _dinv_body,
        grid=(NPAD // BM,),
        in_specs=[pl.BlockSpec((BM, NC), lambda i: (i, 0))],
        out_specs=[pl.BlockSpec((BM, 1), lambda i: (i, 0)),
                   pl.BlockSpec((BM, 1), lambda i: (i, 0))],
        out_shape=[jax.ShapeDtypeStruct((NPAD, 1), jnp.float32),
                   jax.ShapeDtypeStruct((NPAD, 1), jnp.float32)],
    )(deg_t)


def _mm_call(xpad, wt, b2, sv):
    return pl.pallas_call(
        _mm_body,
        grid=(NPAD // BM,),
        in_specs=[pl.BlockSpec((BM, F), lambda i: (i, 0)),
                  pl.BlockSpec((F, F), lambda i: (0, 0)),
                  pl.BlockSpec((1, F), lambda i: (0, 0)),
                  pl.BlockSpec((BM, 1), lambda i: (i, 0))],
        out_specs=pl.BlockSpec((BM, F), lambda i: (i, 0)),
        out_shape=jax.ShapeDtypeStruct((NPAD, F), jnp.float32),
    )(xpad, wt, b2, sv)


def _scale_call(acc_pair, sv):
    return pl.pallas_call(
        _scale_body,
        grid=(NPAD // BM,),
        in_specs=[pl.BlockSpec((NC, BM, F), lambda i: (0, i, 0)),
                  pl.BlockSpec((BM, 1), lambda i: (i, 0))],
        out_specs=pl.BlockSpec((BM, F), lambda i: (i, 0)),
        out_shape=jax.ShapeDtypeStruct((NPAD, F), jnp.float32),
    )(acc_pair, sv)


def kernel(x, edge_index, W, b):
    ei = edge_index.astype(jnp.int32)
    loop = jnp.arange(N, dtype=jnp.int32)
    pad_e = EPAD - E_LOOP
    row = jnp.concatenate([ei[0], loop, jnp.zeros((pad_e,), jnp.int32)])
    col = jnp.concatenate([ei[1], loop, jnp.full((pad_e,), JCOL, jnp.int32)])
    row3 = row.reshape(NT, NCH, C)
    col3 = col.reshape(NT, NCH, C)

    xpad = jnp.zeros((NPAD, F), jnp.float32).at[:N].set(x)
    wt = W.T
    b2 = b.reshape(1, F)
    ones_c = jnp.ones((C,), jnp.float32)
    zero_n = jnp.zeros((NPAD,), jnp.float32)
    zero_rows = jnp.zeros((NPAD, F), jnp.float32)

    deg_pair = _deg_kernel(col3, ones_c, zero_n)
    dinv, dinv2 = _dinv_call(deg_pair.T)

    g = _mm_call(xpad, wt, b2, dinv)
    for hop in range(HOPS):
        acc_pair = _hop_kernel(g, row3, col3, zero_rows)
        sv = dinv if hop == HOPS - 1 else dinv2
        g = _scale_call(acc_pair, sv)
    return g[:N]
